# Initial kernel scaffold; baseline (speedup 1.0000x reference)
import jax
import jax.numpy as jnp
from jax.experimental import pallas as pl


def kernel(x):
    raise NotImplementedError("write your pallas kernel here")


# speedup vs baseline: 1.8220x; 1.8220x over previous
import math
import numpy as np
import jax
import jax.numpy as jnp
from jax.experimental import pallas as pl
from jax.experimental.pallas import tpu as pltpu

_G = math.exp(-1.0 / (2.0 * 1.5 * 1.5))
_K0 = _G / (1.0 + 2.0 * _G)
_K1 = 1.0 / (1.0 + 2.0 * _G)


def _bilinear_matrix(in_size: int, scale: int) -> np.ndarray:
    out_size = in_size * scale
    o = np.arange(out_size, dtype=np.float64)
    src = np.clip((o + 0.5) * (in_size / out_size) - 0.5, 0.0, None)
    i0 = np.minimum(np.floor(src).astype(np.int64), in_size - 1)
    i1 = np.minimum(i0 + 1, in_size - 1)
    wgt = src - i0
    m = np.zeros((out_size, in_size), dtype=np.float64)
    m[np.arange(out_size), i0] += 1.0 - wgt
    m[np.arange(out_size), i1] += wgt
    return m


def _blur_band_matrix(n: int) -> np.ndarray:
    g = np.zeros((n, n), dtype=np.float64)
    for i in range(n):
        for off, kk in ((-1, _K0), (0, _K1), (1, _K0)):
            j = i + off
            if j < 0:
                j = -j
            elif j > n - 1:
                j = 2 * (n - 1) - j
            g[i, j] += kk
    return g


def _make_body(bch: int, pk: int, sh: int, sw: int):
    nq = bch // pk

    def _body(x_ref, r_ref, a_ref, o_ref):
        h = x_ref.shape[1]
        w = x_ref.shape[2]
        xb = x_ref[...].reshape(bch * h, w).astype(jnp.bfloat16)
        t = jnp.dot(xb, r_ref[...], preferred_element_type=jnp.float32)
        t = t.astype(jnp.bfloat16).reshape(nq, pk * h, sw)
        a = a_ref[...]
        for q in range(nq):
            y = jnp.dot(a, t[q], preferred_element_type=jnp.float32)
            o_ref[q * pk:(q + 1) * pk] = y.reshape(pk, sh, sw)

    return _body


def _blur_upsample(x: jax.Array, s: int) -> jax.Array:
    n, c, h, w = x.shape
    nc = n * c
    sh, sw = s * h, s * w

    a_np = _bilinear_matrix(h, s) @ _blur_band_matrix(h)
    r_np = (_bilinear_matrix(w, s) @ _blur_band_matrix(w)).T

    pk = 1
    for cand in (4, 2):
        if nc % cand == 0 and cand * h <= 256:
            pk = cand
            break
    a_bd = np.zeros((pk * sh, pk * h), dtype=np.float64)
    for b in range(pk):
        a_bd[b * sh:(b + 1) * sh, b * h:(b + 1) * h] = a_np
    a_bd = jnp.asarray(a_bd, dtype=jnp.bfloat16)
    r_bf = jnp.asarray(r_np, dtype=jnp.bfloat16)

    bch = pk
    for d in range(nc, 0, -1):
        if nc % d == 0 and d % pk == 0 and d * (h * w + sh * sw) * 4 <= (4 << 20):
            if nc // d >= 8 or d == nc:
                bch = d
                break
    g = nc // bch

    xp = x.reshape(nc, h, w)
    flops = nc * (2 * sh * h * w + 2 * sh * w * sw)
    bytes_accessed = int(xp.size * 4 + nc * sh * sw * 4 + a_bd.size * 2
                         + r_bf.size * 2)

    out = pl.pallas_call(
        _make_body(bch, pk, sh, sw),
        out_shape=jax.ShapeDtypeStruct((nc, sh, sw), x.dtype),
        grid=(g,),
        in_specs=[
            pl.BlockSpec((bch, h, w), lambda i: (i, 0, 0)),
            pl.BlockSpec((w, sw), lambda i: (0, 0),
                         pipeline_mode=pl.Buffered(1)),
            pl.BlockSpec((pk * sh, pk * h), lambda i: (0, 0),
                         pipeline_mode=pl.Buffered(1)),
        ],
        out_specs=pl.BlockSpec((bch, sh, sw), lambda i: (i, 0, 0)),
        compiler_params=pltpu.CompilerParams(dimension_semantics=("parallel",)),
        cost_estimate=pl.CostEstimate(flops=int(flops), transcendentals=0,
                                      bytes_accessed=bytes_accessed),
    )(xp, r_bf, a_bd)

    return out.reshape(n, c, sh, sw)


def kernel(x):
    return _blur_upsample(x, 2)


# speedup vs baseline: 2.2213x; 1.2192x over previous
import math
import numpy as np
import jax
import jax.numpy as jnp
from jax.experimental import pallas as pl
from jax.experimental.pallas import tpu as pltpu

_G = math.exp(-1.0 / (2.0 * 1.5 * 1.5))
_K0 = _G / (1.0 + 2.0 * _G)
_K1 = 1.0 / (1.0 + 2.0 * _G)


def _bilinear_matrix(in_size: int, scale: int) -> np.ndarray:
    out_size = in_size * scale
    o = np.arange(out_size, dtype=np.float64)
    src = np.clip((o + 0.5) * (in_size / out_size) - 0.5, 0.0, None)
    i0 = np.minimum(np.floor(src).astype(np.int64), in_size - 1)
    i1 = np.minimum(i0 + 1, in_size - 1)
    wgt = src - i0
    m = np.zeros((out_size, in_size), dtype=np.float64)
    m[np.arange(out_size), i0] += 1.0 - wgt
    m[np.arange(out_size), i1] += wgt
    return m


def _blur_band_matrix(n: int) -> np.ndarray:
    g = np.zeros((n, n), dtype=np.float64)
    for i in range(n):
        for off, kk in ((-1, _K0), (0, _K1), (1, _K0)):
            j = i + off
            if j < 0:
                j = -j
            elif j > n - 1:
                j = 2 * (n - 1) - j
            g[i, j] += kk
    return g


def _make_body(bch: int, pk: int, sh: int, sw: int):
    nq = bch // pk

    def _body(x_ref, r_ref, a_ref, o_ref):
        h = x_ref.shape[1]
        w = x_ref.shape[2]
        xb = x_ref[...].reshape(bch * h, w).astype(jnp.bfloat16)
        t = jnp.dot(xb, r_ref[...], preferred_element_type=jnp.float32)
        t = t.astype(jnp.bfloat16).reshape(nq, pk * h, sw)
        a = a_ref[...]
        for q in range(nq):
            y = jnp.dot(a, t[q], preferred_element_type=jnp.float32)
            o_ref[q * pk:(q + 1) * pk] = y.reshape(pk, sh, sw)

    return _body


def _blur_upsample(x: jax.Array, s: int) -> jax.Array:
    n, c, h, w = x.shape
    nc = n * c
    sh, sw = s * h, s * w

    a_np = _bilinear_matrix(h, s) @ _blur_band_matrix(h)
    r_np = (_bilinear_matrix(w, s) @ _blur_band_matrix(w)).T

    pk = 1
    for cand in (4, 2):
        if nc % cand == 0 and cand * h <= 256:
            pk = cand
            break
    a_bd = np.zeros((pk * sh, pk * h), dtype=np.float64)
    for b in range(pk):
        a_bd[b * sh:(b + 1) * sh, b * h:(b + 1) * h] = a_np
    a_bd = jnp.asarray(a_bd, dtype=jnp.bfloat16)
    r_bf = jnp.asarray(r_np, dtype=jnp.bfloat16)

    bch = pk
    for d in range(nc, 0, -1):
        if nc % d == 0 and d % pk == 0 and d * (h * w + sh * sw) * 4 <= (16 << 20):
            if nc // d >= 8 or d == nc:
                bch = d
                break
    g = nc // bch

    xp = x.reshape(nc, h, w)
    flops = nc * (2 * sh * h * w + 2 * sh * w * sw)
    bytes_accessed = int(xp.size * 4 + nc * sh * sw * 4 + a_bd.size * 2
                         + r_bf.size * 2)

    out = pl.pallas_call(
        _make_body(bch, pk, sh, sw),
        out_shape=jax.ShapeDtypeStruct((nc, sh, sw), x.dtype),
        grid=(g,),
        in_specs=[
            pl.BlockSpec((bch, h, w), lambda i: (i, 0, 0)),
            pl.BlockSpec((w, sw), lambda i: (0, 0),
                         pipeline_mode=pl.Buffered(1)),
            pl.BlockSpec((pk * sh, pk * h), lambda i: (0, 0),
                         pipeline_mode=pl.Buffered(1)),
        ],
        out_specs=pl.BlockSpec((bch, sh, sw), lambda i: (i, 0, 0)),
        compiler_params=pltpu.CompilerParams(
            dimension_semantics=("arbitrary",)),
        cost_estimate=pl.CostEstimate(flops=int(flops), transcendentals=0,
                                      bytes_accessed=bytes_accessed),
    )(xp, r_bf, a_bd)

    return out.reshape(n, c, sh, sw)


def kernel(x):
    return _blur_upsample(x, 2)
